# baseline (speedup 1.0000x reference)
import functools

import jax
import jax.numpy as jnp
from jax import lax
from jax.experimental import pallas as pl
from jax.experimental.pallas import tpu as pltpu
from jax.experimental.pallas import tpu_sc as plsc

_N_NODES = 10000
_N_EDGES = 320000
_D = 128
_DE = 16
_EPS = 1e-5

_CHUNK = 128
_N_WORKERS = 32
_N_CHUNKS = _N_EDGES // _CHUNK
_TRIPS = _N_CHUNKS // _N_WORKERS
_PAIRS = _TRIPS // 2
_CW = _CHUNK * _DE


def _nodeproj_body(h_ref, ws_ref, wd_ref, hs_ref, hd_ref):
    h = h_ref[...]
    hs_ref[...] = jnp.dot(h, ws_ref[...], preferred_element_type=jnp.float32)
    hd_ref[...] = jnp.dot(h, wd_ref[...], preferred_element_type=jnp.float32)


def _node_proj(h, w_s, w_d):
    return pl.pallas_call(
        _nodeproj_body,
        out_shape=(
            jax.ShapeDtypeStruct((_N_NODES, _DE), jnp.float32),
            jax.ShapeDtypeStruct((_N_NODES, _DE), jnp.float32),
        ),
    )(h, w_s, w_d)


_K2_G = 25


def _detile(etile0, etile1):
    a = jnp.concatenate([etile0, etile1], axis=1)
    b = a.transpose(0, 2, 1)
    b4 = b.reshape(_K2_G, 16, 8, _DE)
    rows = [b4[:, :, j, :] for j in range(8)]
    ee = jnp.concatenate(rows, axis=2)
    return ee.reshape(16 * _K2_G, _D)


def _ew_body(etile_ref, wk_ref, b_ref, out_ref):
    ee = _detile(etile_ref[0], etile_ref[1])
    out_ref[...] = (
        jnp.dot(ee, wk_ref[...], preferred_element_type=jnp.float32)
        + b_ref[...]
    )


def _edge_proj(e_tiles, w_kron, b_tiled):
    grid = _N_CHUNKS // _K2_G
    return pl.pallas_call(
        _ew_body,
        grid=(grid,),
        in_specs=[
            pl.BlockSpec((2, _K2_G, 8, _D), lambda i: (0, i, 0, 0)),
            pl.BlockSpec((_D, _D), lambda i: (0, 0)),
            pl.BlockSpec((1, _D), lambda i: (0, 0)),
        ],
        out_specs=pl.BlockSpec((16 * _K2_G, _D), lambda i: (i, 0)),
        out_shape=jax.ShapeDtypeStruct((_N_EDGES * _DE // _D, _D), jnp.float32),
    )(e_tiles, w_kron, b_tiled)


def _sc_body(hs_hbm, hd_hbm, ewf_hbm, ei_hbm, zeros_hbm,
             elin_hbm, stats_hbm, cnt_hbm,
             idx_s, idx_d, ewv, hsv, hdv, elv, onesv, statv, cnt_sh,
             sem_in, sem_g, sem_w, sem_sc):
    c = lax.axis_index("c")
    s = lax.axis_index("s")
    wid = s * 2 + c

    @pl.when(s == 0)
    def _init():
        pltpu.sync_copy(zeros_hbm, cnt_sh)

    for i in range(_CHUNK // 16):
        onesv[pl.ds(16 * i, 16)] = jnp.full((16,), 1.0, jnp.float32)
    plsc.subcore_barrier()

    def issue_in(chunk, b):
        base = chunk * _CHUNK
        pltpu.async_copy(ei_hbm.at[0, pl.ds(base, _CHUNK)], idx_s[b], sem_in[b])
        pltpu.async_copy(ei_hbm.at[1, pl.ds(base, _CHUNK)], idx_d[b], sem_in[b])
        pltpu.async_copy(ewf_hbm.at[pl.ds(chunk * _CW, _CW)], ewv[b], sem_in[b])

    def wait_in(b):
        pltpu.make_async_copy(ei_hbm.at[0, pl.ds(0, _CHUNK)], idx_s[b], sem_in[b]).wait()
        pltpu.make_async_copy(ei_hbm.at[1, pl.ds(0, _CHUNK)], idx_d[b], sem_in[b]).wait()
        pltpu.make_async_copy(ewf_hbm.at[pl.ds(0, _CW)], ewv[b], sem_in[b]).wait()

    def issue_gathers(b):
        pltpu.async_copy(hs_hbm.at[idx_s[b]], hsv[b], sem_g[b])
        pltpu.async_copy(hd_hbm.at[idx_d[b]], hdv[b], sem_g[b])

    def wait_gathers(b):
        pltpu.make_async_copy(hs_hbm.at[idx_s[b]], hsv[b], sem_g[b]).wait()
        pltpu.make_async_copy(hd_hbm.at[idx_d[b]], hdv[b], sem_g[b]).wait()

    def issue_scatter(b):
        pltpu.async_copy(onesv, cnt_sh.at[idx_d[b]], sem_sc[b], add=True)

    def wait_scatter(b):
        pltpu.make_async_copy(onesv, cnt_sh.at[idx_d[b]], sem_sc[b]).wait()

    def issue_wb(chunk, b):
        pltpu.async_copy(elv[b], elin_hbm.at[pl.ds(chunk * _CW, _CW)], sem_w[b])

    def wait_wb(b):
        pltpu.make_async_copy(elv[b], elin_hbm.at[pl.ds(0, _CW)], sem_w[b]).wait()

    def compute(b, carry, mval):
        def row(i, cc):
            s1, s2 = cc
            v = hsv[b][i] + hdv[b][i] + ewv[b][pl.ds(i * _DE, _DE)]
            elv[b][pl.ds(i * _DE, _DE)] = v
            vm = v * mval
            return (s1 + vm, s2 + vm * v)

        return lax.fori_loop(0, _CHUNK, row, carry)

    issue_in(wid, 0)
    one = jnp.float32(1.0)
    echunk = _TRIPS * _N_WORKERS + jnp.minimum(wid, 3)

    def pair(j, carry):
        t0 = 2 * j
        wait_in(0)
        issue_gathers(0)
        issue_scatter(0)

        @pl.when(j > 0)
        def _():
            wait_wb(1)
            wait_scatter(1)

        issue_in(wid + (t0 + 1) * _N_WORKERS, 1)
        wait_gathers(0)
        wait_in(1)
        issue_gathers(1)
        issue_scatter(1)

        @pl.when(j > 0)
        def _():
            wait_wb(0)

        carry = compute(0, carry, one)
        issue_wb(wid + t0 * _N_WORKERS, 0)
        wait_scatter(0)
        nxt = jnp.where(t0 + 2 < _TRIPS, wid + (t0 + 2) * _N_WORKERS, echunk)
        issue_in(nxt, 0)
        wait_gathers(1)
        carry = compute(1, carry, one)
        issue_wb(wid + (t0 + 1) * _N_WORKERS, 1)
        return carry

    z = jnp.zeros((16,), jnp.float32)
    ssum, ssq = lax.fori_loop(0, _PAIRS, pair, (z, z))

    wait_scatter(1)
    mval = jnp.where(wid < 4, 1.0, 0.0).astype(jnp.float32)
    for i in range(_CHUNK // 16):
        onesv[pl.ds(16 * i, 16)] = jax.lax.broadcast(mval, (16,))
    wait_in(0)
    issue_gathers(0)
    issue_scatter(0)
    wait_wb(0)
    wait_gathers(0)
    ssum, ssq = compute(0, (ssum, ssq), mval)
    issue_wb(echunk, 0)
    wait_wb(0)
    wait_scatter(0)
    wait_wb(1)

    statv[pl.ds(0, 16)] = ssum
    statv[pl.ds(16, 16)] = ssq
    pltpu.sync_copy(statv, stats_hbm.at[wid])

    plsc.subcore_barrier()

    @pl.when(s == 0)
    def _flush():
        pltpu.sync_copy(cnt_sh, cnt_hbm.at[c])


def _sc_edge_kernel(hs, hd, ew_flat, edge_index, zeros):
    mesh = plsc.VectorSubcoreMesh(core_axis_name="c", subcore_axis_name="s")
    dbl = lambda ty: [ty, ty]
    f = functools.partial(
        pl.kernel,
        mesh=mesh,
        compiler_params=pltpu.CompilerParams(use_tc_tiling_on_sc=False,
                                            needs_layout_passes=False),
        out_type=(
            jax.ShapeDtypeStruct((_N_EDGES * _DE,), jnp.float32),
            jax.ShapeDtypeStruct((_N_WORKERS, 2 * _DE), jnp.float32),
            jax.ShapeDtypeStruct((2, _N_NODES), jnp.float32),
        ),
        scratch_types=[
            dbl(pltpu.VMEM((_CHUNK,), jnp.int32)),
            dbl(pltpu.VMEM((_CHUNK,), jnp.int32)),
            dbl(pltpu.VMEM((_CW,), jnp.float32)),
            dbl(pltpu.VMEM((_CHUNK, _DE), jnp.float32)),
            dbl(pltpu.VMEM((_CHUNK, _DE), jnp.float32)),
            dbl(pltpu.VMEM((_CW,), jnp.float32)),
            pltpu.VMEM((_CHUNK,), jnp.float32),
            pltpu.VMEM((2 * _DE,), jnp.float32),
            pltpu.VMEM_SHARED((_N_NODES,), jnp.float32),
            dbl(pltpu.SemaphoreType.DMA),
            dbl(pltpu.SemaphoreType.DMA),
            dbl(pltpu.SemaphoreType.DMA),
            dbl(pltpu.SemaphoreType.DMA),
        ],
    )(_sc_body)
    return f(hs, hd, ew_flat, edge_index, zeros)


def _e2_body(stats_ref, g_ref, bt_ref, elin_ref, e_ref, out_ref):
    st = stats_ref[...]
    ssum = jnp.sum(st, axis=0, keepdims=True)
    inv_n = 1.0 / _N_EDGES
    mu = ssum[:, :_DE] * inv_n
    msq = ssum[:, _DE:] * inv_n
    rstd = lax.rsqrt(msq - mu * mu + _EPS)
    mu8 = jnp.concatenate([mu] * 8, axis=1)
    rstd8 = jnp.concatenate([rstd] * 8, axis=1)
    ee = _detile(e_ref[0], e_ref[1])
    x = (elin_ref[...] - mu8) * rstd8 * g_ref[...] + bt_ref[...]
    out_ref[...] = ee + x * jax.nn.sigmoid(x)


def _e2_apply(stats, g_tiled, bt_tiled, elin_resh, e_tiles):
    grid = _N_CHUNKS // _K2_G
    return pl.pallas_call(
        _e2_body,
        grid=(grid,),
        in_specs=[
            pl.BlockSpec((_N_WORKERS, 2 * _DE), lambda i: (0, 0)),
            pl.BlockSpec((1, _D), lambda i: (0, 0)),
            pl.BlockSpec((1, _D), lambda i: (0, 0)),
            pl.BlockSpec((16 * _K2_G, _D), lambda i: (i, 0)),
            pl.BlockSpec((2, _K2_G, 8, _D), lambda i: (0, i, 0, 0)),
        ],
        out_specs=pl.BlockSpec((16 * _K2_G, _D), lambda i: (i, 0)),
        out_shape=jax.ShapeDtypeStruct((_N_EDGES * _DE // _D, _D), jnp.float32),
    )(stats, g_tiled, bt_tiled, elin_resh, e_tiles)


def _node_body(h_ref, wgd_ref, bgd_ref, wgs_ref, bgs_ref, c0_ref, c1_ref,
               gg_ref, btg_ref, wl_ref, bl_ref, out_ref):
    h = h_ref[...]
    mask = ((c0_ref[...] + c1_ref[...]) > 0.0).astype(jnp.float32)
    gl = jnp.dot(h, wgd_ref[...], preferred_element_type=jnp.float32) + bgd_ref[...]
    pre = (jnp.dot(h, wgs_ref[...], preferred_element_type=jnp.float32)
           + bgs_ref[...] + gl * mask)
    mu = jnp.mean(pre, axis=0, keepdims=True)
    d = pre - mu
    var = jnp.mean(d * d, axis=0, keepdims=True)
    xn = d * lax.rsqrt(var + _EPS) * gg_ref[...] + btg_ref[...]
    h2 = xn * jax.nn.sigmoid(xn) + h
    out_ref[...] = (jnp.dot(h2, wl_ref[...], preferred_element_type=jnp.float32)
                    + bl_ref[...])


def _node_update(h, w_gdst, b_gdst, w_gsrc, b_gsrc, c0, c1, g_gate, bt_gate,
                 w_lin, b_lin):
    return pl.pallas_call(
        _node_body,
        out_shape=jax.ShapeDtypeStruct((_N_NODES, _D), jnp.float32),
    )(h, w_gdst, b_gdst, w_gsrc, b_gsrc, c0, c1, g_gate, bt_gate, w_lin, b_lin)


def kernel(h, e, edge_index, W_upd, b_upd, g_upd, bt_upd, W_act, b_act,
           W_gdst, b_gdst, W_gsrc, b_gsrc, g_gate, bt_gate, W_lin, b_lin):
    ei = edge_index.astype(jnp.int32)

    w_s = W_upd[:_D]
    w_d = W_upd[_D:2 * _D]
    w_e = W_upd[2 * _D:]
    w_kron = jnp.kron(jnp.eye(8, dtype=jnp.float32), w_e)
    b_tiled = jnp.tile(b_upd.reshape(1, _DE), (1, 8))
    g_tiled = jnp.tile(g_upd.reshape(1, _DE), (1, 8))
    btu_tiled = jnp.tile(bt_upd.reshape(1, _DE), (1, 8))

    e_tiles = e.T.reshape(2, 8, _N_CHUNKS, _D).transpose(0, 2, 1, 3)

    hs, hd = _node_proj(h, w_s, w_d)
    ew_resh = _edge_proj(e_tiles, w_kron, b_tiled)
    ew_flat = ew_resh.reshape(_N_EDGES * _DE)

    zeros = jnp.zeros((_N_NODES,), jnp.float32)
    e_lin, stats, cnt = _sc_edge_kernel(hs, hd, ew_flat, ei, zeros)

    elin_resh = e_lin.reshape(_N_EDGES * _DE // _D, _D)
    e2 = _e2_apply(stats, g_tiled, btu_tiled, elin_resh, e_tiles)
    e2 = e2.reshape(_N_EDGES, _DE)

    c0 = cnt[0].reshape(_N_NODES, 1)
    c1 = cnt[1].reshape(_N_NODES, 1)
    h2 = _node_update(h, W_gdst, b_gdst.reshape(1, _D),
                      W_gsrc, b_gsrc.reshape(1, _D), c0, c1,
                      g_gate.reshape(1, _D), bt_gate.reshape(1, _D),
                      W_lin, b_lin.reshape(1, _D))
    return (h2, e2)


# speedup vs baseline: 1.1701x; 1.1701x over previous
import functools

import jax
import jax.numpy as jnp
from jax import lax
from jax.experimental import pallas as pl
from jax.experimental.pallas import tpu as pltpu
from jax.experimental.pallas import tpu_sc as plsc

_N_NODES = 10000
_N_EDGES = 320000
_D = 128
_DE = 16
_EPS = 1e-5

_CHUNK = 128
_N_WORKERS = 32
_N_CHUNKS = _N_EDGES // _CHUNK
_TRIPS = _N_CHUNKS // _N_WORKERS
_PAIRS = _TRIPS // 2
_CW = _CHUNK * _DE


def _nodeproj_body(h_ref, ws_ref, wd_ref, hs_ref, hd_ref):
    h = h_ref[...]
    hs_ref[...] = jnp.dot(h, ws_ref[...], preferred_element_type=jnp.float32)
    hd_ref[...] = jnp.dot(h, wd_ref[...], preferred_element_type=jnp.float32)


def _node_proj(h, w_s, w_d):
    return pl.pallas_call(
        _nodeproj_body,
        out_shape=(
            jax.ShapeDtypeStruct((_N_NODES, _DE), jnp.float32),
            jax.ShapeDtypeStruct((_N_NODES, _DE), jnp.float32),
        ),
    )(h, w_s, w_d)


def _ew_body(e_ref, wk_ref, b_ref, out_ref):
    out_ref[...] = (
        jnp.dot(e_ref[...], wk_ref[...], preferred_element_type=jnp.float32)
        + b_ref[...]
    )


def _edge_proj(e_resh, w_kron, b_tiled):
    rows = e_resh.shape[0]
    blk = 4000
    grid = rows // blk
    return pl.pallas_call(
        _ew_body,
        grid=(grid,),
        in_specs=[
            pl.BlockSpec((blk, _D), lambda i: (i, 0)),
            pl.BlockSpec((_D, _D), lambda i: (0, 0)),
            pl.BlockSpec((1, _D), lambda i: (0, 0)),
        ],
        out_specs=pl.BlockSpec((blk, _D), lambda i: (i, 0)),
        out_shape=jax.ShapeDtypeStruct((rows, _D), jnp.float32),
    )(e_resh, w_kron, b_tiled)


def _sc_body(hs_hbm, hd_hbm, ewf_hbm, ei_hbm, zeros_hbm,
             elin_hbm, stats_hbm, cnt_hbm,
             idx_s, idx_d, ewv, hsv, hdv, elv, onesv, statv, cnt_sh,
             sem_in, sem_g, sem_w, sem_sc):
    c = lax.axis_index("c")
    s = lax.axis_index("s")
    wid = s * 2 + c

    @pl.when(s == 0)
    def _init():
        pltpu.sync_copy(zeros_hbm, cnt_sh)

    for i in range(_CHUNK // 16):
        onesv[pl.ds(16 * i, 16)] = jnp.full((16,), 1.0, jnp.float32)
    plsc.subcore_barrier()

    def issue_in(chunk, b):
        base = chunk * _CHUNK
        pltpu.async_copy(ei_hbm.at[0, pl.ds(base, _CHUNK)], idx_s[b], sem_in[b])
        pltpu.async_copy(ei_hbm.at[1, pl.ds(base, _CHUNK)], idx_d[b], sem_in[b])
        pltpu.async_copy(ewf_hbm.at[pl.ds(chunk * _CW, _CW)], ewv[b], sem_in[b])

    def wait_in(b):
        pltpu.make_async_copy(ei_hbm.at[0, pl.ds(0, _CHUNK)], idx_s[b], sem_in[b]).wait()
        pltpu.make_async_copy(ei_hbm.at[1, pl.ds(0, _CHUNK)], idx_d[b], sem_in[b]).wait()
        pltpu.make_async_copy(ewf_hbm.at[pl.ds(0, _CW)], ewv[b], sem_in[b]).wait()

    def issue_gathers(b):
        pltpu.async_copy(hs_hbm.at[idx_s[b]], hsv[b], sem_g[b])
        pltpu.async_copy(hd_hbm.at[idx_d[b]], hdv[b], sem_g[b])

    def wait_gathers(b):
        pltpu.make_async_copy(hs_hbm.at[idx_s[b]], hsv[b], sem_g[b]).wait()
        pltpu.make_async_copy(hd_hbm.at[idx_d[b]], hdv[b], sem_g[b]).wait()

    def issue_scatter(b):
        pltpu.async_copy(onesv, cnt_sh.at[idx_d[b]], sem_sc[b], add=True)

    def wait_scatter(b):
        pltpu.make_async_copy(onesv, cnt_sh.at[idx_d[b]], sem_sc[b]).wait()

    def issue_wb(chunk, b):
        pltpu.async_copy(elv[b], elin_hbm.at[pl.ds(chunk * _CW, _CW)], sem_w[b])

    def wait_wb(b):
        pltpu.make_async_copy(elv[b], elin_hbm.at[pl.ds(0, _CW)], sem_w[b]).wait()

    def compute(b, carry, mval):
        def row(i, cc):
            s1, s2 = cc
            v = hsv[b][i] + hdv[b][i] + ewv[b][pl.ds(i * _DE, _DE)]
            elv[b][pl.ds(i * _DE, _DE)] = v
            vm = v * mval
            return (s1 + vm, s2 + vm * v)

        return lax.fori_loop(0, _CHUNK, row, carry)

    issue_in(wid, 0)
    one = jnp.float32(1.0)
    echunk = _TRIPS * _N_WORKERS + jnp.minimum(wid, 3)

    def pair(j, carry):
        t0 = 2 * j
        wait_in(0)
        issue_gathers(0)
        issue_scatter(0)

        @pl.when(j > 0)
        def _():
            wait_wb(1)
            wait_scatter(1)

        issue_in(wid + (t0 + 1) * _N_WORKERS, 1)
        wait_gathers(0)
        wait_in(1)
        issue_gathers(1)
        issue_scatter(1)

        @pl.when(j > 0)
        def _():
            wait_wb(0)

        carry = compute(0, carry, one)
        issue_wb(wid + t0 * _N_WORKERS, 0)
        wait_scatter(0)
        nxt = jnp.where(t0 + 2 < _TRIPS, wid + (t0 + 2) * _N_WORKERS, echunk)
        issue_in(nxt, 0)
        wait_gathers(1)
        carry = compute(1, carry, one)
        issue_wb(wid + (t0 + 1) * _N_WORKERS, 1)
        return carry

    z = jnp.zeros((16,), jnp.float32)
    ssum, ssq = lax.fori_loop(0, _PAIRS, pair, (z, z))

    wait_scatter(1)
    mval = jnp.where(wid < 4, 1.0, 0.0).astype(jnp.float32)
    for i in range(_CHUNK // 16):
        onesv[pl.ds(16 * i, 16)] = jax.lax.broadcast(mval, (16,))
    wait_in(0)
    issue_gathers(0)
    issue_scatter(0)
    wait_wb(0)
    wait_gathers(0)
    ssum, ssq = compute(0, (ssum, ssq), mval)
    issue_wb(echunk, 0)
    wait_wb(0)
    wait_scatter(0)
    wait_wb(1)

    statv[pl.ds(0, 16)] = ssum
    statv[pl.ds(16, 16)] = ssq
    pltpu.sync_copy(statv, stats_hbm.at[wid])

    plsc.subcore_barrier()

    @pl.when(s == 0)
    def _flush():
        pltpu.sync_copy(cnt_sh, cnt_hbm.at[c])


def _sc_edge_kernel(hs, hd, ew_flat, edge_index, zeros):
    mesh = plsc.VectorSubcoreMesh(core_axis_name="c", subcore_axis_name="s")
    dbl = lambda ty: [ty, ty]
    f = functools.partial(
        pl.kernel,
        mesh=mesh,
        compiler_params=pltpu.CompilerParams(use_tc_tiling_on_sc=False,
                                            needs_layout_passes=False),
        out_type=(
            jax.ShapeDtypeStruct((_N_EDGES * _DE,), jnp.float32),
            jax.ShapeDtypeStruct((_N_WORKERS, 2 * _DE), jnp.float32),
            jax.ShapeDtypeStruct((2, _N_NODES), jnp.float32),
        ),
        scratch_types=[
            dbl(pltpu.VMEM((_CHUNK,), jnp.int32)),
            dbl(pltpu.VMEM((_CHUNK,), jnp.int32)),
            dbl(pltpu.VMEM((_CW,), jnp.float32)),
            dbl(pltpu.VMEM((_CHUNK, _DE), jnp.float32)),
            dbl(pltpu.VMEM((_CHUNK, _DE), jnp.float32)),
            dbl(pltpu.VMEM((_CW,), jnp.float32)),
            pltpu.VMEM((_CHUNK,), jnp.float32),
            pltpu.VMEM((2 * _DE,), jnp.float32),
            pltpu.VMEM_SHARED((_N_NODES,), jnp.float32),
            dbl(pltpu.SemaphoreType.DMA),
            dbl(pltpu.SemaphoreType.DMA),
            dbl(pltpu.SemaphoreType.DMA),
            dbl(pltpu.SemaphoreType.DMA),
        ],
    )(_sc_body)
    return f(hs, hd, ew_flat, edge_index, zeros)


def _e2_body(stats_ref, g_ref, bt_ref, elin_ref, e_ref, out_ref):
    st = stats_ref[...]
    ssum = jnp.sum(st, axis=0, keepdims=True)
    inv_n = 1.0 / _N_EDGES
    mu = ssum[:, :_DE] * inv_n
    msq = ssum[:, _DE:] * inv_n
    rstd = lax.rsqrt(msq - mu * mu + _EPS)
    mu8 = jnp.concatenate([mu] * 8, axis=1)
    rstd8 = jnp.concatenate([rstd] * 8, axis=1)
    x = (elin_ref[...] - mu8) * rstd8 * g_ref[...] + bt_ref[...]
    out_ref[...] = e_ref[...] + x * jax.nn.sigmoid(x)


def _e2_apply(stats, g_tiled, bt_tiled, elin_resh, e_resh):
    rows = e_resh.shape[0]
    blk = 4000
    grid = rows // blk
    return pl.pallas_call(
        _e2_body,
        grid=(grid,),
        in_specs=[
            pl.BlockSpec((_N_WORKERS, 2 * _DE), lambda i: (0, 0)),
            pl.BlockSpec((1, _D), lambda i: (0, 0)),
            pl.BlockSpec((1, _D), lambda i: (0, 0)),
            pl.BlockSpec((blk, _D), lambda i: (i, 0)),
            pl.BlockSpec((blk, _D), lambda i: (i, 0)),
        ],
        out_specs=pl.BlockSpec((blk, _D), lambda i: (i, 0)),
        out_shape=jax.ShapeDtypeStruct((rows, _D), jnp.float32),
    )(stats, g_tiled, bt_tiled, elin_resh, e_resh)


def _node_body(h_ref, wgd_ref, bgd_ref, wgs_ref, bgs_ref, c0_ref, c1_ref,
               gg_ref, btg_ref, wl_ref, bl_ref, out_ref):
    h = h_ref[...]
    mask = ((c0_ref[...] + c1_ref[...]) > 0.0).astype(jnp.float32)
    gl = jnp.dot(h, wgd_ref[...], preferred_element_type=jnp.float32) + bgd_ref[...]
    pre = (jnp.dot(h, wgs_ref[...], preferred_element_type=jnp.float32)
           + bgs_ref[...] + gl * mask)
    mu = jnp.mean(pre, axis=0, keepdims=True)
    d = pre - mu
    var = jnp.mean(d * d, axis=0, keepdims=True)
    xn = d * lax.rsqrt(var + _EPS) * gg_ref[...] + btg_ref[...]
    h2 = xn * jax.nn.sigmoid(xn) + h
    out_ref[...] = (jnp.dot(h2, wl_ref[...], preferred_element_type=jnp.float32)
                    + bl_ref[...])


def _node_update(h, w_gdst, b_gdst, w_gsrc, b_gsrc, c0, c1, g_gate, bt_gate,
                 w_lin, b_lin):
    return pl.pallas_call(
        _node_body,
        out_shape=jax.ShapeDtypeStruct((_N_NODES, _D), jnp.float32),
    )(h, w_gdst, b_gdst, w_gsrc, b_gsrc, c0, c1, g_gate, bt_gate, w_lin, b_lin)


def kernel(h, e, edge_index, W_upd, b_upd, g_upd, bt_upd, W_act, b_act,
           W_gdst, b_gdst, W_gsrc, b_gsrc, g_gate, bt_gate, W_lin, b_lin):
    ei = edge_index.astype(jnp.int32)

    w_s = W_upd[:_D]
    w_d = W_upd[_D:2 * _D]
    w_e = W_upd[2 * _D:]
    w_kron = jnp.kron(jnp.eye(8, dtype=jnp.float32), w_e)
    b_tiled = jnp.tile(b_upd.reshape(1, _DE), (1, 8))
    g_tiled = jnp.tile(g_upd.reshape(1, _DE), (1, 8))
    btu_tiled = jnp.tile(bt_upd.reshape(1, _DE), (1, 8))

    e_resh = e.reshape(_N_EDGES * _DE // _D, _D)

    hs, hd = _node_proj(h, w_s, w_d)
    ew_resh = _edge_proj(e_resh, w_kron, b_tiled)
    ew_flat = ew_resh.reshape(_N_EDGES * _DE)

    zeros = jnp.zeros((_N_NODES,), jnp.float32)
    e_lin, stats, cnt = _sc_edge_kernel(hs, hd, ew_flat, ei, zeros)

    elin_resh = e_lin.reshape(_N_EDGES * _DE // _D, _D)
    e2 = _e2_apply(stats, g_tiled, btu_tiled, elin_resh, e_resh)
    e2 = e2.reshape(_N_EDGES, _DE)

    c0 = cnt[0].reshape(_N_NODES, 1)
    c1 = cnt[1].reshape(_N_NODES, 1)
    h2 = _node_update(h, W_gdst, b_gdst.reshape(1, _D),
                      W_gsrc, b_gsrc.reshape(1, _D), c0, c1,
                      g_gate.reshape(1, _D), bt_gate.reshape(1, _D),
                      W_lin, b_lin.reshape(1, _D))
    return (h2, e2)


# speedup vs baseline: 1.1860x; 1.0135x over previous
import functools

import jax
import jax.numpy as jnp
from jax import lax
from jax.experimental import pallas as pl
from jax.experimental.pallas import tpu as pltpu
from jax.experimental.pallas import tpu_sc as plsc

_N_NODES = 10000
_N_EDGES = 320000
_D = 128
_DE = 16
_EPS = 1e-5

_CHUNK = 128
_N_WORKERS = 32
_N_CHUNKS = _N_EDGES // _CHUNK
_TRIPS = _N_CHUNKS // _N_WORKERS
_PAIRS = _TRIPS // 2
_CW = _CHUNK * _DE


def _nodeproj_body(h_ref, ws_ref, wd_ref, hs_ref, hd_ref):
    h = h_ref[...]
    hs_ref[...] = jnp.dot(h, ws_ref[...], preferred_element_type=jnp.float32)
    hd_ref[...] = jnp.dot(h, wd_ref[...], preferred_element_type=jnp.float32)


def _node_proj(h, w_s, w_d):
    return pl.pallas_call(
        _nodeproj_body,
        out_shape=(
            jax.ShapeDtypeStruct((_N_NODES, _DE), jnp.float32),
            jax.ShapeDtypeStruct((_N_NODES, _DE), jnp.float32),
        ),
    )(h, w_s, w_d)


def _ew_body(e_ref, wk_ref, b_ref, out_ref):
    out_ref[...] = (
        jnp.dot(e_ref[...], wk_ref[...], preferred_element_type=jnp.float32)
        + b_ref[...]
    )


def _edge_proj(e_resh, w_kron, b_tiled):
    rows = e_resh.shape[0]
    blk = 4000
    grid = rows // blk
    return pl.pallas_call(
        _ew_body,
        grid=(grid,),
        in_specs=[
            pl.BlockSpec((blk, _D), lambda i: (i, 0)),
            pl.BlockSpec((_D, _D), lambda i: (0, 0)),
            pl.BlockSpec((1, _D), lambda i: (0, 0)),
        ],
        out_specs=pl.BlockSpec((blk, _D), lambda i: (i, 0)),
        out_shape=jax.ShapeDtypeStruct((rows, _D), jnp.float32),
    )(e_resh, w_kron, b_tiled)


def _sc_body(hs_hbm, hd_hbm, ewf_hbm, ei_hbm, zeros_hbm,
             elin_hbm, stats_hbm, cnt_hbm,
             idx_s, idx_d, ewv, hsv, hdv, elv, onesv, statv, cnt_sh,
             sem_in, sem_g, sem_w, sem_sc):
    c = lax.axis_index("c")
    s = lax.axis_index("s")
    wid = s * 2 + c

    @pl.when(s == 0)
    def _init():
        pltpu.sync_copy(zeros_hbm, cnt_sh)

    for i in range(_CHUNK // 16):
        onesv[pl.ds(16 * i, 16)] = jnp.full((16,), 1.0, jnp.float32)
    plsc.subcore_barrier()

    def issue_in(chunk, b):
        base = chunk * _CHUNK
        pltpu.async_copy(ei_hbm.at[0, pl.ds(base, _CHUNK)], idx_s[b], sem_in[b])
        pltpu.async_copy(ei_hbm.at[1, pl.ds(base, _CHUNK)], idx_d[b], sem_in[b])
        pltpu.async_copy(ewf_hbm.at[pl.ds(chunk * _CW, _CW)], ewv[b], sem_in[b])

    def wait_in(b):
        pltpu.make_async_copy(ei_hbm.at[0, pl.ds(0, _CHUNK)], idx_s[b], sem_in[b]).wait()
        pltpu.make_async_copy(ei_hbm.at[1, pl.ds(0, _CHUNK)], idx_d[b], sem_in[b]).wait()
        pltpu.make_async_copy(ewf_hbm.at[pl.ds(0, _CW)], ewv[b], sem_in[b]).wait()

    def issue_gathers(b):
        pltpu.async_copy(hs_hbm.at[idx_s[b]], hsv[b], sem_g[b])
        pltpu.async_copy(hd_hbm.at[idx_d[b]], hdv[b], sem_g[b])

    def wait_gathers(b):
        pltpu.make_async_copy(hs_hbm.at[idx_s[b]], hsv[b], sem_g[b]).wait()
        pltpu.make_async_copy(hd_hbm.at[idx_d[b]], hdv[b], sem_g[b]).wait()

    def issue_scatter(b):
        pltpu.async_copy(onesv, cnt_sh.at[idx_d[b]], sem_sc[b], add=True)

    def wait_scatter(b):
        pltpu.make_async_copy(onesv, cnt_sh.at[idx_d[b]], sem_sc[b]).wait()

    def issue_wb(chunk, b):
        pltpu.async_copy(elv[b], elin_hbm.at[pl.ds(chunk * _CW, _CW)], sem_w[b])

    def wait_wb(b):
        pltpu.make_async_copy(elv[b], elin_hbm.at[pl.ds(0, _CW)], sem_w[b]).wait()

    def compute(b, carry):
        def rows(i, cc):
            s1, s2 = cc
            i0 = 2 * i
            v = hsv[b][i0] + hdv[b][i0] + ewv[b][pl.ds(i0 * _DE, _DE)]
            elv[b][pl.ds(i0 * _DE, _DE)] = v
            i1 = i0 + 1
            w = hsv[b][i1] + hdv[b][i1] + ewv[b][pl.ds(i1 * _DE, _DE)]
            elv[b][pl.ds(i1 * _DE, _DE)] = w
            return (s1 + v + w, s2 + v * v + w * w)

        return lax.fori_loop(0, _CHUNK // 2, rows, carry)

    def compute_masked(b, carry, mval):
        def row(i, cc):
            s1, s2 = cc
            v = hsv[b][i] + hdv[b][i] + ewv[b][pl.ds(i * _DE, _DE)]
            elv[b][pl.ds(i * _DE, _DE)] = v
            vm = v * mval
            return (s1 + vm, s2 + vm * v)

        return lax.fori_loop(0, _CHUNK, row, carry)

    issue_in(wid, 0)
    echunk = _TRIPS * _N_WORKERS + jnp.minimum(wid, 3)

    def pair(j, carry):
        t0 = 2 * j
        wait_in(0)
        issue_gathers(0)
        issue_scatter(0)

        @pl.when(j > 0)
        def _():
            wait_wb(1)
            wait_scatter(1)

        issue_in(wid + (t0 + 1) * _N_WORKERS, 1)
        wait_gathers(0)
        wait_in(1)
        issue_gathers(1)
        issue_scatter(1)

        @pl.when(j > 0)
        def _():
            wait_wb(0)

        carry = compute(0, carry)
        issue_wb(wid + t0 * _N_WORKERS, 0)
        wait_scatter(0)
        nxt = jnp.where(t0 + 2 < _TRIPS, wid + (t0 + 2) * _N_WORKERS, echunk)
        issue_in(nxt, 0)
        wait_gathers(1)
        carry = compute(1, carry)
        issue_wb(wid + (t0 + 1) * _N_WORKERS, 1)
        return carry

    z = jnp.zeros((16,), jnp.float32)
    ssum, ssq = lax.fori_loop(0, _PAIRS, pair, (z, z))

    wait_scatter(1)
    mval = jnp.where(wid < 4, 1.0, 0.0).astype(jnp.float32)
    for i in range(_CHUNK // 16):
        onesv[pl.ds(16 * i, 16)] = jax.lax.broadcast(mval, (16,))
    wait_in(0)
    issue_gathers(0)
    issue_scatter(0)
    wait_wb(0)
    wait_gathers(0)
    ssum, ssq = compute_masked(0, (ssum, ssq), mval)
    issue_wb(echunk, 0)
    wait_wb(0)
    wait_scatter(0)
    wait_wb(1)

    statv[pl.ds(0, 16)] = ssum
    statv[pl.ds(16, 16)] = ssq
    pltpu.sync_copy(statv, stats_hbm.at[wid])

    plsc.subcore_barrier()

    @pl.when(s == 0)
    def _flush():
        pltpu.sync_copy(cnt_sh, cnt_hbm.at[c])


def _sc_edge_kernel(hs, hd, ew_flat, edge_index, zeros):
    mesh = plsc.VectorSubcoreMesh(core_axis_name="c", subcore_axis_name="s")
    dbl = lambda ty: [ty, ty]
    f = functools.partial(
        pl.kernel,
        mesh=mesh,
        compiler_params=pltpu.CompilerParams(use_tc_tiling_on_sc=False,
                                            needs_layout_passes=False),
        out_type=(
            jax.ShapeDtypeStruct((_N_EDGES * _DE,), jnp.float32),
            jax.ShapeDtypeStruct((_N_WORKERS, 2 * _DE), jnp.float32),
            jax.ShapeDtypeStruct((2, _N_NODES), jnp.float32),
        ),
        scratch_types=[
            dbl(pltpu.VMEM((_CHUNK,), jnp.int32)),
            dbl(pltpu.VMEM((_CHUNK,), jnp.int32)),
            dbl(pltpu.VMEM((_CW,), jnp.float32)),
            dbl(pltpu.VMEM((_CHUNK, _DE), jnp.float32)),
            dbl(pltpu.VMEM((_CHUNK, _DE), jnp.float32)),
            dbl(pltpu.VMEM((_CW,), jnp.float32)),
            pltpu.VMEM((_CHUNK,), jnp.float32),
            pltpu.VMEM((2 * _DE,), jnp.float32),
            pltpu.VMEM_SHARED((_N_NODES,), jnp.float32),
            dbl(pltpu.SemaphoreType.DMA),
            dbl(pltpu.SemaphoreType.DMA),
            dbl(pltpu.SemaphoreType.DMA),
            dbl(pltpu.SemaphoreType.DMA),
        ],
    )(_sc_body)
    return f(hs, hd, ew_flat, edge_index, zeros)


def _e2_body(stats_ref, g_ref, bt_ref, elin_ref, e_ref, out_ref):
    st = stats_ref[...]
    ssum = jnp.sum(st, axis=0, keepdims=True)
    inv_n = 1.0 / _N_EDGES
    mu = ssum[:, :_DE] * inv_n
    msq = ssum[:, _DE:] * inv_n
    rstd = lax.rsqrt(msq - mu * mu + _EPS)
    mu8 = jnp.concatenate([mu] * 8, axis=1)
    rstd8 = jnp.concatenate([rstd] * 8, axis=1)
    x = (elin_ref[...] - mu8) * rstd8 * g_ref[...] + bt_ref[...]
    out_ref[...] = e_ref[...] + x * jax.nn.sigmoid(x)


def _e2_apply(stats, g_tiled, bt_tiled, elin_resh, e_resh):
    rows = e_resh.shape[0]
    blk = 4000
    grid = rows // blk
    return pl.pallas_call(
        _e2_body,
        grid=(grid,),
        in_specs=[
            pl.BlockSpec((_N_WORKERS, 2 * _DE), lambda i: (0, 0)),
            pl.BlockSpec((1, _D), lambda i: (0, 0)),
            pl.BlockSpec((1, _D), lambda i: (0, 0)),
            pl.BlockSpec((blk, _D), lambda i: (i, 0)),
            pl.BlockSpec((blk, _D), lambda i: (i, 0)),
        ],
        out_specs=pl.BlockSpec((blk, _D), lambda i: (i, 0)),
        out_shape=jax.ShapeDtypeStruct((rows, _D), jnp.float32),
    )(stats, g_tiled, bt_tiled, elin_resh, e_resh)


def _node_body(h_ref, wgd_ref, bgd_ref, wgs_ref, bgs_ref, c0_ref, c1_ref,
               gg_ref, btg_ref, wl_ref, bl_ref, out_ref):
    h = h_ref[...]
    mask = ((c0_ref[...] + c1_ref[...]) > 0.0).astype(jnp.float32)
    gl = jnp.dot(h, wgd_ref[...], preferred_element_type=jnp.float32) + bgd_ref[...]
    pre = (jnp.dot(h, wgs_ref[...], preferred_element_type=jnp.float32)
           + bgs_ref[...] + gl * mask)
    mu = jnp.mean(pre, axis=0, keepdims=True)
    d = pre - mu
    var = jnp.mean(d * d, axis=0, keepdims=True)
    xn = d * lax.rsqrt(var + _EPS) * gg_ref[...] + btg_ref[...]
    h2 = xn * jax.nn.sigmoid(xn) + h
    out_ref[...] = (jnp.dot(h2, wl_ref[...], preferred_element_type=jnp.float32)
                    + bl_ref[...])


def _node_update(h, w_gdst, b_gdst, w_gsrc, b_gsrc, c0, c1, g_gate, bt_gate,
                 w_lin, b_lin):
    return pl.pallas_call(
        _node_body,
        out_shape=jax.ShapeDtypeStruct((_N_NODES, _D), jnp.float32),
    )(h, w_gdst, b_gdst, w_gsrc, b_gsrc, c0, c1, g_gate, bt_gate, w_lin, b_lin)


def kernel(h, e, edge_index, W_upd, b_upd, g_upd, bt_upd, W_act, b_act,
           W_gdst, b_gdst, W_gsrc, b_gsrc, g_gate, bt_gate, W_lin, b_lin):
    ei = edge_index.astype(jnp.int32)

    w_s = W_upd[:_D]
    w_d = W_upd[_D:2 * _D]
    w_e = W_upd[2 * _D:]
    w_kron = jnp.kron(jnp.eye(8, dtype=jnp.float32), w_e)
    b_tiled = jnp.tile(b_upd.reshape(1, _DE), (1, 8))
    g_tiled = jnp.tile(g_upd.reshape(1, _DE), (1, 8))
    btu_tiled = jnp.tile(bt_upd.reshape(1, _DE), (1, 8))

    e_resh = e.reshape(_N_EDGES * _DE // _D, _D)

    hs, hd = _node_proj(h, w_s, w_d)
    ew_resh = _edge_proj(e_resh, w_kron, b_tiled)
    ew_flat = ew_resh.reshape(_N_EDGES * _DE)

    zeros = jnp.zeros((_N_NODES,), jnp.float32)
    e_lin, stats, cnt = _sc_edge_kernel(hs, hd, ew_flat, ei, zeros)

    elin_resh = e_lin.reshape(_N_EDGES * _DE // _D, _D)
    e2 = _e2_apply(stats, g_tiled, btu_tiled, elin_resh, e_resh)
    e2 = e2.reshape(_N_EDGES, _DE)

    c0 = cnt[0].reshape(_N_NODES, 1)
    c1 = cnt[1].reshape(_N_NODES, 1)
    h2 = _node_update(h, W_gdst, b_gdst.reshape(1, _D),
                      W_gsrc, b_gsrc.reshape(1, _D), c0, c1,
                      g_gate.reshape(1, _D), bt_gate.reshape(1, _D),
                      W_lin, b_lin.reshape(1, _D))
    return (h2, e2)
